# baseline (speedup 1.0000x reference)
import functools

import jax
import jax.numpy as jnp
from jax import lax
from jax.experimental import pallas as pl
from jax.experimental.pallas import tpu as pltpu
from jax.experimental.pallas import tpu_sc as plsc

N_NODES = 10000
DIM_IN = 128
DIM_H = 64
DIM_O = 6
N_GRAPH = 64
N_EDGE = 320000

NC, NS, LANES = 2, 16, 16
NW = NC * NS
NP = 10240
ROWS_W = NP // NS
CH = 128
NCH_W = 81
NCH_TOT = NW * NCH_W
EP = NCH_TOT * CH
NB = 4
BLK = 640
BLK2 = BLK // 2
GRID = NP // BLK

_mesh = plsc.VectorSubcoreMesh(
    core_axis_name="c", subcore_axis_name="s", num_cores=NC, num_subcores=NS)
_sc_params = pltpu.CompilerParams(use_tc_tiling_on_sc=False)

NCH_E = N_EDGE // CH
W_SPLIT = NCH_E // NCH_W
N_REAL_SPLIT = NCH_E - W_SPLIT * NCH_W
NCH_AUX = NCH_TOT - NCH_E


def _load_idx(e_hbm, aux_hbm, idx, w):
    @pl.when(w < W_SPLIT)
    def _():
        pltpu.sync_copy(e_hbm.at[pl.ds(w * NCH_W, NCH_W)], idx)

    @pl.when(w == W_SPLIT)
    def _():
        pltpu.sync_copy(e_hbm.at[pl.ds(W_SPLIT * NCH_W, N_REAL_SPLIT)],
                        idx.at[pl.ds(0, N_REAL_SPLIT)])
        pltpu.sync_copy(aux_hbm.at[pl.ds(0, NCH_W - N_REAL_SPLIT)],
                        idx.at[pl.ds(N_REAL_SPLIT, NCH_W - N_REAL_SPLIT)])

    @pl.when(w == W_SPLIT + 1)
    def _():
        pltpu.sync_copy(aux_hbm.at[pl.ds(NCH_W - N_REAL_SPLIT, NCH_W)], idx)


@functools.partial(
    pl.kernel,
    out_type=jax.ShapeDtypeStruct((NC * NP, LANES), jnp.float32),
    mesh=_mesh,
    scratch_types=[
        pltpu.VMEM((CH, LANES), jnp.float32),
        pltpu.VMEM((CH, LANES), jnp.float32),
        pltpu.VMEM((NCH_W, CH), jnp.int32),
        pltpu.SemaphoreType.DMA,
        pltpu.VMEM_SHARED((NP, LANES), jnp.float32),
    ],
    compiler_params=_sc_params,
)
def _deg_kernel(dst_hbm, aux_hbm, out_hbm, zb, ones_v, didx, ssem, cnt_sp):
    c = lax.axis_index("c")
    s = lax.axis_index("s")
    w = c * NS + s

    def fill(i, _):
        zb[i, :] = jnp.zeros((LANES,), jnp.float32)
        ones_v[i, :] = jnp.ones((LANES,), jnp.float32)
        return 0

    lax.fori_loop(0, CH, fill, 0)
    _load_idx(dst_hbm, aux_hbm, didx, w)
    for k in range(ROWS_W // CH):
        pltpu.sync_copy(zb, cnt_sp.at[pl.ds(s * ROWS_W + k * CH, CH)])
    plsc.subcore_barrier()

    def pair(i, _):
        for k in range(2):
            j = i * 2 + k
            pltpu.async_copy(ones_v, cnt_sp.at[didx.at[j]], ssem, add=True)

            @pl.when(j >= 4)
            def _():
                pltpu.make_async_copy(ones_v, cnt_sp.at[didx.at[j]],
                                      ssem).wait()
        return 0

    lax.fori_loop(0, (NCH_W - 1) // 2, pair, 0)
    pltpu.async_copy(ones_v, cnt_sp.at[didx.at[NCH_W - 1]], ssem, add=True)
    for _ in range(5):
        pltpu.make_async_copy(ones_v, cnt_sp.at[didx.at[0]], ssem).wait()
    plsc.subcore_barrier()
    pltpu.sync_copy(cnt_sp.at[pl.ds(s * ROWS_W, ROWS_W)],
                    out_hbm.at[pl.ds(c * NP + s * ROWS_W, ROWS_W)])


@functools.partial(
    pl.kernel,
    out_type=jax.ShapeDtypeStruct((NC * NP, DIM_H), jnp.float32),
    mesh=_mesh,
    scratch_types=[
        pltpu.VMEM((CH, DIM_H), jnp.float32),
        pltpu.VMEM((NB, CH, DIM_H), jnp.float32),
        pltpu.VMEM((NCH_W, CH), jnp.int32),
        pltpu.VMEM((NCH_W, CH), jnp.int32),
        pltpu.SemaphoreType.DMA,
        pltpu.VMEM_SHARED((NP, DIM_H), jnp.float32),
    ],
    compiler_params=_sc_params,
)
def _agg_kernel(g_hbm, src_hbm, dst_hbm, aux_hbm, out_hbm, zb, rows, sidx,
                didx, gsem, acc_sp):
    c = lax.axis_index("c")
    s = lax.axis_index("s")
    w = c * NS + s

    def fill(i, _):
        for k in range(DIM_H // LANES):
            zb[i, pl.ds(k * LANES, LANES)] = jnp.zeros((LANES,), jnp.float32)
        return 0

    lax.fori_loop(0, CH, fill, 0)
    _load_idx(src_hbm, aux_hbm, sidx, w)
    _load_idx(dst_hbm, aux_hbm, didx, w)
    for k in range(ROWS_W // CH):
        pltpu.sync_copy(zb, acc_sp.at[pl.ds(s * ROWS_W + k * CH, CH)])
    plsc.subcore_barrier()

    for b in range(NB - 1):
        pltpu.async_copy(g_hbm.at[sidx.at[b]], rows.at[b], gsem)

    def quad(i, _):
        for k in range(NB):
            j = i * NB + k
            pltpu.make_async_copy(g_hbm.at[sidx.at[k]], rows.at[k],
                                  gsem).wait()
            jn = j + NB - 1

            @pl.when(jn < NCH_W)
            def _():
                bn = (k + NB - 1) % NB
                pltpu.async_copy(g_hbm.at[sidx.at[jn]], rows.at[bn], gsem)

            pltpu.sync_copy(rows.at[k], acc_sp.at[didx.at[j]], add=True)
        return 0

    lax.fori_loop(0, (NCH_W - 1) // NB, quad, 0)
    pltpu.make_async_copy(g_hbm.at[sidx.at[0]], rows.at[0], gsem).wait()
    pltpu.sync_copy(rows.at[0], acc_sp.at[didx.at[NCH_W - 1]], add=True)
    plsc.subcore_barrier()
    pltpu.sync_copy(acc_sp.at[pl.ds(s * ROWS_W, ROWS_W)],
                    out_hbm.at[pl.ds(c * NP + s * ROWS_W, ROWS_W)])


GRIDB = 8
BB2 = NP // 2 // GRIDB
BLKB = 2 * BB2


def _scales(dd):
    i = pl.program_id(0)
    rn = lax.broadcasted_iota(jnp.int32, (BB2, BLKB // 8), 0)
    rc = lax.broadcasted_iota(jnp.int32, (BB2, BLKB // 8), 1)
    u2 = (rc == rn // 4).astype(jnp.float32)
    t2 = jnp.dot(u2, dd, preferred_element_type=jnp.float32)
    jm = lax.broadcasted_iota(jnp.int32, (BB2, 128), 0) % 4
    lane = lax.broadcasted_iota(jnp.int32, (BB2, 128), 1)
    degp = jnp.zeros((BB2, 128), jnp.float32)
    for q in range(4):
        half = jnp.where(lane < DIM_H,
                         jnp.broadcast_to(t2[:, 32 * q:32 * q + 1],
                                          (BB2, 128)),
                         jnp.broadcast_to(t2[:, 32 * q + 16:32 * q + 17],
                                          (BB2, 128)))
        degp = jnp.where(jm == q, half, degp)
    j_glob = i * BB2 + lax.broadcasted_iota(jnp.int32, (BB2, 128), 0)
    node = 2 * j_glob + (lane >= DIM_H).astype(jnp.int32)
    ok = (node < N_NODES) & (degp > 0.0)
    return jnp.where(ok, lax.rsqrt(jnp.maximum(degp, 1e-30)), 0.0)


def _tc1_body(xe_ref, xo_ref, w1_ref, m_ref):
    lo = jnp.dot(xe_ref[...], w1_ref[...], preferred_element_type=jnp.float32)
    hi = jnp.dot(xo_ref[...], w1_ref[...], preferred_element_type=jnp.float32)
    m_ref[...] = jnp.concatenate([lo, hi], axis=1)


def _tc1(xe, xo, W1):
    return pl.pallas_call(
        _tc1_body,
        grid=(GRIDB,),
        in_specs=[
            pl.BlockSpec((BB2, DIM_IN), lambda i: (i, 0)),
            pl.BlockSpec((BB2, DIM_IN), lambda i: (i, 0)),
            pl.BlockSpec((DIM_IN, DIM_H), lambda i: (0, 0)),
        ],
        out_specs=pl.BlockSpec((BB2, 128), lambda i: (i, 0)),
        out_shape=jax.ShapeDtypeStruct((NP // 2, 128), jnp.float32),
    )(xe, xo, W1)


def _tcd_body(m_ref, deg_ref, sc_ref, g_ref):
    dd = deg_ref[0] + deg_ref[1]
    sc2 = _scales(dd)
    sc_ref[...] = sc2
    g_ref[...] = m_ref[...] * sc2


def _tcd(m1p, deg2):
    return pl.pallas_call(
        _tcd_body,
        grid=(GRIDB,),
        in_specs=[
            pl.BlockSpec((BB2, 128), lambda i: (i, 0)),
            pl.BlockSpec((NC, BLKB // 8, 128), lambda i: (0, i, 0)),
        ],
        out_specs=[
            pl.BlockSpec((BB2, 128), lambda i: (i, 0)),
            pl.BlockSpec((BB2, 128), lambda i: (i, 0)),
        ],
        out_shape=[
            jax.ShapeDtypeStruct((NP // 2, 128), jnp.float32),
            jax.ShapeDtypeStruct((NP // 2, 128), jnp.float32),
        ],
    )(m1p, deg2)


def _tc2_body(a_ref, sc_ref, b1_ref, w2_ref, g_ref):
    sc2 = sc_ref[...]
    a = (a_ref[0] + a_ref[1]) * sc2 + b1_ref[...]
    h = jnp.maximum(a, 0.0)
    g_ref[...] = jnp.dot(h, w2_ref[...],
                         preferred_element_type=jnp.float32) * sc2


def _tc2(a1, scp, b1p, W2blk):
    return pl.pallas_call(
        _tc2_body,
        grid=(GRIDB,),
        in_specs=[
            pl.BlockSpec((NC, BB2, 128), lambda i: (0, i, 0)),
            pl.BlockSpec((BB2, 128), lambda i: (i, 0)),
            pl.BlockSpec((1, 128), lambda i: (0, 0)),
            pl.BlockSpec((128, 128), lambda i: (0, 0)),
        ],
        out_specs=pl.BlockSpec((BB2, 128), lambda i: (i, 0)),
        out_shape=jax.ShapeDtypeStruct((NP // 2, 128), jnp.float32),
    )(a1, scp, b1p, W2blk)


def _tc3_body(a_ref, sc_ref, b2_ref, bt_ref, wl_ref, bl_ref, fin_ref, acc):
    i = pl.program_id(0)
    sc2 = sc_ref[...]
    h2p = jnp.maximum((a_ref[0] + a_ref[1]) * sc2 + b2_ref[...], 0.0)
    h2cat = jnp.concatenate([h2p[:, :DIM_H], h2p[:, DIM_H:]], axis=0)
    bt = bt_ref[0]
    gids = lax.broadcasted_iota(jnp.int32, (N_GRAPH, BLKB), 0)
    oh = (bt == gids).astype(jnp.float32)
    haug = jnp.concatenate([h2cat, jnp.ones((BLKB, DIM_H), jnp.float32)],
                           axis=1)
    part = jnp.dot(oh, haug, preferred_element_type=jnp.float32)

    @pl.when(i == 0)
    def _():
        acc[...] = part

    @pl.when(i > 0)
    def _():
        acc[...] += part

    @pl.when(i == GRIDB - 1)
    def _():
        sums = acc[:, :DIM_H]
        cnt = acc[:, DIM_H:DIM_H + 1]
        pooled = sums / jnp.maximum(cnt, 1.0)
        fin_ref[...] = jnp.dot(pooled, wl_ref[...],
                               preferred_element_type=jnp.float32) + bl_ref[...]


def _tc3(a2, scp, b2p, batchp, wlp, blp):
    return pl.pallas_call(
        _tc3_body,
        grid=(GRIDB,),
        in_specs=[
            pl.BlockSpec((NC, BB2, 128), lambda i: (0, i, 0)),
            pl.BlockSpec((BB2, 128), lambda i: (i, 0)),
            pl.BlockSpec((1, 128), lambda i: (0, 0)),
            pl.BlockSpec((1, 1, BLKB), lambda i: (i, 0, 0)),
            pl.BlockSpec((DIM_H, 128), lambda i: (0, 0)),
            pl.BlockSpec((1, 128), lambda i: (0, 0)),
        ],
        out_specs=pl.BlockSpec((N_GRAPH, 128), lambda i: (0, 0)),
        out_shape=jax.ShapeDtypeStruct((N_GRAPH, 128), jnp.float32),
        scratch_shapes=[pltpu.VMEM((N_GRAPH, 128), jnp.float32)],
    )(a2, scp, b2p, batchp, wlp, blp)


def kernel(x, edge_index, batch, W1, b1, W2, b2, Wlin, blin):
    loop = jnp.arange(N_NODES, dtype=jnp.int32)
    npad = EP - (N_EDGE + N_NODES)
    pad_rows = N_NODES + (jnp.arange(npad, dtype=jnp.int32) % (NP - N_NODES))
    aux = jnp.concatenate([loop, pad_rows]).reshape(NCH_AUX, CH)
    e0 = edge_index[0].reshape(NCH_E, CH)
    e1 = edge_index[1].reshape(NCH_E, CH)

    deg2 = _deg_kernel(e1, aux).reshape(NC, NP // 8, 128)

    xp = jnp.pad(x, ((0, NP - N_NODES), (0, 0)))
    xe, xo = xp[0::2], xp[1::2]
    m1p = _tc1(xe, xo, W1)
    scp, g1 = _tcd(m1p, deg2)
    a1 = _agg_kernel(g1.reshape(NP, DIM_H), e0, e1, aux
                     ).reshape(NC, NP // 2, 128)
    b1p = jnp.concatenate([b1, b1]).reshape(1, 128)
    W2blk = jnp.zeros((128, 128), W2.dtype)
    W2blk = W2blk.at[:DIM_H, :DIM_H].set(W2).at[DIM_H:, DIM_H:].set(W2)
    g2 = _tc2(a1, scp, b1p, W2blk)
    a2 = _agg_kernel(g2.reshape(NP, DIM_H), e0, e1, aux
                     ).reshape(NC, NP // 2, 128)

    batchp = jnp.pad(batch, (0, NP - N_NODES), constant_values=N_GRAPH
                     ).reshape(GRIDB, BB2, 2).transpose(0, 2, 1
                     ).reshape(GRIDB, 1, BLKB)
    b2p = jnp.concatenate([b2, b2]).reshape(1, 128)
    wlp = jnp.pad(Wlin, ((0, 0), (0, 128 - DIM_O)))
    blp = jnp.pad(blin, (0, 128 - DIM_O)).reshape(1, 128)
    fin = _tc3(a2, scp, b2p, batchp, wlp, blp)
    return fin[:, :DIM_O]


# speedup vs baseline: 1.0298x; 1.0298x over previous
import functools

import jax
import jax.numpy as jnp
from jax import lax
from jax.experimental import pallas as pl
from jax.experimental.pallas import tpu as pltpu
from jax.experimental.pallas import tpu_sc as plsc

N_NODES = 10000
DIM_IN = 128
DIM_H = 64
DIM_O = 6
N_GRAPH = 64
N_EDGE = 320000

NC, NS, LANES = 2, 16, 16
NW = NC * NS
NP = 10240
ROWS_W = NP // NS
CH = 128
NCH_W = 81
NCH_TOT = NW * NCH_W
EP = NCH_TOT * CH
NB = 4
BLK = 640
BLK2 = BLK // 2
GRID = NP // BLK

_mesh = plsc.VectorSubcoreMesh(
    core_axis_name="c", subcore_axis_name="s", num_cores=NC, num_subcores=NS)
_sc_params = pltpu.CompilerParams(use_tc_tiling_on_sc=False)

NCH_E = N_EDGE // CH
W_SPLIT = NCH_E // NCH_W
N_REAL_SPLIT = NCH_E - W_SPLIT * NCH_W
NCH_AUX = NCH_TOT - NCH_E


def _load_idx(e_hbm, aux_hbm, idx, w):
    @pl.when(w < W_SPLIT)
    def _():
        pltpu.sync_copy(e_hbm.at[pl.ds(w * NCH_W, NCH_W)], idx)

    @pl.when(w == W_SPLIT)
    def _():
        pltpu.sync_copy(e_hbm.at[pl.ds(W_SPLIT * NCH_W, N_REAL_SPLIT)],
                        idx.at[pl.ds(0, N_REAL_SPLIT)])
        pltpu.sync_copy(aux_hbm.at[pl.ds(0, NCH_W - N_REAL_SPLIT)],
                        idx.at[pl.ds(N_REAL_SPLIT, NCH_W - N_REAL_SPLIT)])

    @pl.when(w == W_SPLIT + 1)
    def _():
        pltpu.sync_copy(aux_hbm.at[pl.ds(NCH_W - N_REAL_SPLIT, NCH_W)], idx)


@functools.partial(
    pl.kernel,
    out_type=jax.ShapeDtypeStruct((NC * NP, LANES), jnp.float32),
    mesh=_mesh,
    scratch_types=[
        pltpu.VMEM((CH, LANES), jnp.float32),
        pltpu.VMEM((CH, LANES), jnp.float32),
        pltpu.VMEM((NCH_W, CH), jnp.int32),
        pltpu.SemaphoreType.DMA,
        pltpu.VMEM_SHARED((NP, LANES), jnp.float32),
    ],
    compiler_params=_sc_params,
)
def _deg_kernel(dst_hbm, aux_hbm, out_hbm, zb, ones_v, didx, ssem, cnt_sp):
    c = lax.axis_index("c")
    s = lax.axis_index("s")
    w = c * NS + s

    def fill(i, _):
        zb[i, :] = jnp.zeros((LANES,), jnp.float32)
        ones_v[i, :] = jnp.ones((LANES,), jnp.float32)
        return 0

    lax.fori_loop(0, CH, fill, 0)
    _load_idx(dst_hbm, aux_hbm, didx, w)
    for k in range(ROWS_W // CH):
        pltpu.sync_copy(zb, cnt_sp.at[pl.ds(s * ROWS_W + k * CH, CH)])
    plsc.subcore_barrier()

    def pair(i, _):
        for k in range(2):
            j = i * 2 + k
            pltpu.async_copy(ones_v, cnt_sp.at[didx.at[j]], ssem, add=True)

            @pl.when(j >= 4)
            def _():
                pltpu.make_async_copy(ones_v, cnt_sp.at[didx.at[j]],
                                      ssem).wait()
        return 0

    lax.fori_loop(0, (NCH_W - 1) // 2, pair, 0)
    pltpu.async_copy(ones_v, cnt_sp.at[didx.at[NCH_W - 1]], ssem, add=True)
    for _ in range(5):
        pltpu.make_async_copy(ones_v, cnt_sp.at[didx.at[0]], ssem).wait()
    plsc.subcore_barrier()
    pltpu.sync_copy(cnt_sp.at[pl.ds(s * ROWS_W, ROWS_W)],
                    out_hbm.at[pl.ds(c * NP + s * ROWS_W, ROWS_W)])


@functools.partial(
    pl.kernel,
    out_type=jax.ShapeDtypeStruct((NC * NP, DIM_H), jnp.float32),
    mesh=_mesh,
    scratch_types=[
        pltpu.VMEM((CH, DIM_H), jnp.float32),
        pltpu.VMEM((NB, CH, DIM_H), jnp.float32),
        pltpu.VMEM((NCH_W, CH), jnp.int32),
        pltpu.VMEM((NCH_W, CH), jnp.int32),
        pltpu.SemaphoreType.DMA,
        pltpu.VMEM_SHARED((NP, DIM_H), jnp.float32),
    ],
    compiler_params=_sc_params,
)
def _agg_kernel(g_hbm, src_hbm, dst_hbm, aux_hbm, out_hbm, zb, rows, sidx,
                didx, gsem, acc_sp):
    c = lax.axis_index("c")
    s = lax.axis_index("s")
    w = c * NS + s

    def fill(i, _):
        for k in range(DIM_H // LANES):
            zb[i, pl.ds(k * LANES, LANES)] = jnp.zeros((LANES,), jnp.float32)
        return 0

    lax.fori_loop(0, CH, fill, 0)
    _load_idx(src_hbm, aux_hbm, sidx, w)
    _load_idx(dst_hbm, aux_hbm, didx, w)
    for k in range(ROWS_W // CH):
        pltpu.sync_copy(zb, acc_sp.at[pl.ds(s * ROWS_W + k * CH, CH)])
    plsc.subcore_barrier()

    for b in range(NB - 1):
        pltpu.async_copy(g_hbm.at[sidx.at[b]], rows.at[b], gsem)

    def quad(i, _):
        for k in range(NB):
            j = i * NB + k
            pltpu.make_async_copy(g_hbm.at[sidx.at[k]], rows.at[k],
                                  gsem).wait()
            jn = j + NB - 1

            @pl.when(jn < NCH_W)
            def _():
                bn = (k + NB - 1) % NB
                pltpu.async_copy(g_hbm.at[sidx.at[jn]], rows.at[bn], gsem)

            pltpu.sync_copy(rows.at[k], acc_sp.at[didx.at[j]], add=True)
        return 0

    lax.fori_loop(0, (NCH_W - 1) // NB, quad, 0)
    pltpu.make_async_copy(g_hbm.at[sidx.at[0]], rows.at[0], gsem).wait()
    pltpu.sync_copy(rows.at[0], acc_sp.at[didx.at[NCH_W - 1]], add=True)
    plsc.subcore_barrier()
    pltpu.sync_copy(acc_sp.at[pl.ds(s * ROWS_W, ROWS_W)],
                    out_hbm.at[pl.ds(c * NP + s * ROWS_W, ROWS_W)])


GRIDB = 8
BB2 = NP // 2 // GRIDB
BLKB = 2 * BB2


def _scales(dd):
    i = pl.program_id(0)
    rn = lax.broadcasted_iota(jnp.int32, (BB2, BLKB // 8), 0)
    rc = lax.broadcasted_iota(jnp.int32, (BB2, BLKB // 8), 1)
    u2 = (rc == rn // 4).astype(jnp.float32)
    t2 = jnp.dot(u2, dd, preferred_element_type=jnp.float32)
    jm = lax.broadcasted_iota(jnp.int32, (BB2, 128), 0) % 4
    lane = lax.broadcasted_iota(jnp.int32, (BB2, 128), 1)
    degp = jnp.zeros((BB2, 128), jnp.float32)
    for q in range(4):
        half = jnp.where(lane < DIM_H,
                         jnp.broadcast_to(t2[:, 32 * q:32 * q + 1],
                                          (BB2, 128)),
                         jnp.broadcast_to(t2[:, 32 * q + 16:32 * q + 17],
                                          (BB2, 128)))
        degp = jnp.where(jm == q, half, degp)
    j_glob = i * BB2 + lax.broadcasted_iota(jnp.int32, (BB2, 128), 0)
    node = 2 * j_glob + (lane >= DIM_H).astype(jnp.int32)
    ok = (node < N_NODES) & (degp > 0.0)
    return jnp.where(ok, lax.rsqrt(jnp.maximum(degp, 1e-30)), 0.0)


def _tc1_body(x_ref, w1_ref, se_ref, so_ref, m_ref):
    g = jnp.dot(x_ref[...], w1_ref[...], preferred_element_type=jnp.float32)
    lo = jnp.dot(se_ref[...], g, preferred_element_type=jnp.float32)
    hi = jnp.dot(so_ref[...], g, preferred_element_type=jnp.float32)
    m_ref[...] = jnp.concatenate([lo, hi], axis=1)


def _tc1(xp, W1, sep, sop):
    return pl.pallas_call(
        _tc1_body,
        grid=(GRIDB,),
        in_specs=[
            pl.BlockSpec((BLKB, DIM_IN), lambda i: (i, 0)),
            pl.BlockSpec((DIM_IN, DIM_H), lambda i: (0, 0)),
            pl.BlockSpec((BB2, BLKB), lambda i: (0, 0)),
            pl.BlockSpec((BB2, BLKB), lambda i: (0, 0)),
        ],
        out_specs=pl.BlockSpec((BB2, 128), lambda i: (i, 0)),
        out_shape=jax.ShapeDtypeStruct((NP // 2, 128), jnp.float32),
    )(xp, W1, sep, sop)


def _tcd_body(m_ref, deg_ref, sc_ref, g_ref):
    dd = deg_ref[0] + deg_ref[1]
    sc2 = _scales(dd)
    sc_ref[...] = sc2
    g_ref[...] = m_ref[...] * sc2


def _tcd(m1p, deg2):
    return pl.pallas_call(
        _tcd_body,
        grid=(GRIDB,),
        in_specs=[
            pl.BlockSpec((BB2, 128), lambda i: (i, 0)),
            pl.BlockSpec((NC, BLKB // 8, 128), lambda i: (0, i, 0)),
        ],
        out_specs=[
            pl.BlockSpec((BB2, 128), lambda i: (i, 0)),
            pl.BlockSpec((BB2, 128), lambda i: (i, 0)),
        ],
        out_shape=[
            jax.ShapeDtypeStruct((NP // 2, 128), jnp.float32),
            jax.ShapeDtypeStruct((NP // 2, 128), jnp.float32),
        ],
    )(m1p, deg2)


def _tc2_body(a_ref, sc_ref, b1_ref, w2_ref, g_ref):
    sc2 = sc_ref[...]
    a = (a_ref[0] + a_ref[1]) * sc2 + b1_ref[...]
    h = jnp.maximum(a, 0.0)
    g_ref[...] = jnp.dot(h, w2_ref[...],
                         preferred_element_type=jnp.float32) * sc2


def _tc2(a1, scp, b1p, W2blk):
    return pl.pallas_call(
        _tc2_body,
        grid=(GRIDB,),
        in_specs=[
            pl.BlockSpec((NC, BB2, 128), lambda i: (0, i, 0)),
            pl.BlockSpec((BB2, 128), lambda i: (i, 0)),
            pl.BlockSpec((1, 128), lambda i: (0, 0)),
            pl.BlockSpec((128, 128), lambda i: (0, 0)),
        ],
        out_specs=pl.BlockSpec((BB2, 128), lambda i: (i, 0)),
        out_shape=jax.ShapeDtypeStruct((NP // 2, 128), jnp.float32),
    )(a1, scp, b1p, W2blk)


def _tc3_body(a_ref, sc_ref, b2_ref, bt_ref, wl_ref, bl_ref, fin_ref, acc):
    i = pl.program_id(0)
    sc2 = sc_ref[...]
    h2p = jnp.maximum((a_ref[0] + a_ref[1]) * sc2 + b2_ref[...], 0.0)
    h2cat = jnp.concatenate([h2p[:, :DIM_H], h2p[:, DIM_H:]], axis=0)
    bt = bt_ref[0]
    gids = lax.broadcasted_iota(jnp.int32, (N_GRAPH, BLKB), 0)
    oh = (bt == gids).astype(jnp.float32)
    haug = jnp.concatenate([h2cat, jnp.ones((BLKB, DIM_H), jnp.float32)],
                           axis=1)
    part = jnp.dot(oh, haug, preferred_element_type=jnp.float32)

    @pl.when(i == 0)
    def _():
        acc[...] = part

    @pl.when(i > 0)
    def _():
        acc[...] += part

    @pl.when(i == GRIDB - 1)
    def _():
        sums = acc[:, :DIM_H]
        cnt = acc[:, DIM_H:DIM_H + 1]
        pooled = sums / jnp.maximum(cnt, 1.0)
        fin_ref[...] = jnp.dot(pooled, wl_ref[...],
                               preferred_element_type=jnp.float32) + bl_ref[...]


def _tc3(a2, scp, b2p, batchp, wlp, blp):
    return pl.pallas_call(
        _tc3_body,
        grid=(GRIDB,),
        in_specs=[
            pl.BlockSpec((NC, BB2, 128), lambda i: (0, i, 0)),
            pl.BlockSpec((BB2, 128), lambda i: (i, 0)),
            pl.BlockSpec((1, 128), lambda i: (0, 0)),
            pl.BlockSpec((1, 1, BLKB), lambda i: (i, 0, 0)),
            pl.BlockSpec((DIM_H, 128), lambda i: (0, 0)),
            pl.BlockSpec((1, 128), lambda i: (0, 0)),
        ],
        out_specs=pl.BlockSpec((N_GRAPH, 128), lambda i: (0, 0)),
        out_shape=jax.ShapeDtypeStruct((N_GRAPH, 128), jnp.float32),
        scratch_shapes=[pltpu.VMEM((N_GRAPH, 128), jnp.float32)],
    )(a2, scp, b2p, batchp, wlp, blp)


def kernel(x, edge_index, batch, W1, b1, W2, b2, Wlin, blin):
    loop = jnp.arange(N_NODES, dtype=jnp.int32)
    npad = EP - (N_EDGE + N_NODES)
    pad_rows = N_NODES + (jnp.arange(npad, dtype=jnp.int32) % (NP - N_NODES))
    aux = jnp.concatenate([loop, pad_rows]).reshape(NCH_AUX, CH)
    e0 = edge_index[0].reshape(NCH_E, CH)
    e1 = edge_index[1].reshape(NCH_E, CH)

    deg2 = _deg_kernel(e1, aux).reshape(NC, NP // 8, 128)

    xp = jnp.pad(x, ((0, NP - N_NODES), (0, 0)))
    r2 = lax.broadcasted_iota(jnp.int32, (BB2, BLKB), 0)
    c2 = lax.broadcasted_iota(jnp.int32, (BB2, BLKB), 1)
    sep = (c2 == 2 * r2).astype(jnp.float32)
    sop = (c2 == 2 * r2 + 1).astype(jnp.float32)
    m1p = _tc1(xp, W1, sep, sop)
    scp, g1 = _tcd(m1p, deg2)
    a1 = _agg_kernel(g1.reshape(NP, DIM_H), e0, e1, aux
                     ).reshape(NC, NP // 2, 128)
    b1p = jnp.concatenate([b1, b1]).reshape(1, 128)
    W2blk = jnp.zeros((128, 128), W2.dtype)
    W2blk = W2blk.at[:DIM_H, :DIM_H].set(W2).at[DIM_H:, DIM_H:].set(W2)
    g2 = _tc2(a1, scp, b1p, W2blk)
    a2 = _agg_kernel(g2.reshape(NP, DIM_H), e0, e1, aux
                     ).reshape(NC, NP // 2, 128)

    batchp = jnp.pad(batch, (0, NP - N_NODES), constant_values=N_GRAPH
                     ).reshape(GRIDB, BB2, 2).transpose(0, 2, 1
                     ).reshape(GRIDB, 1, BLKB)
    b2p = jnp.concatenate([b2, b2]).reshape(1, 128)
    wlp = jnp.pad(Wlin, ((0, 0), (0, 128 - DIM_O)))
    blp = jnp.pad(blin, (0, 128 - DIM_O)).reshape(1, 128)
    fin = _tc3(a2, scp, b2p, batchp, wlp, blp)
    return fin[:, :DIM_O]
